# baseline (speedup 1.0000x reference)
import functools

import jax
import jax.numpy as jnp
from jax import lax
from jax.experimental import pallas as pl
from jax.experimental.pallas import tpu as pltpu

MAX_NUNITS = 65536
N_DIMS = 512
PHI = 1.5
KK = int(MAX_NUNITS * 0.01)

BM = 8192
ROWS = MAX_NUNITS // 128


def _act_body(x_ref, attn_ref, units_ref, act_ref):
    u = units_ref[...]
    d = x_ref[...] - u
    s = jnp.sum(attn_ref[...] * d * d, axis=1, keepdims=True)
    act_ref[...] = jnp.exp(-s).reshape(BM // 128, 128)


def _select_body(y_true_ref, act_ref, active_ref, w0_ref, w1_ref, out_ref):
    act = act_ref[...]
    active = active_ref[...]
    w0 = w0_ref[...]
    w1 = w1_ref[...]

    def _b(v):
        return v.astype(jnp.bfloat16).astype(jnp.float32)

    am_b = _b(act * active)
    w0_b = _b(w0)
    w1_b = _b(w1)
    y0 = PHI * jnp.sum(am_b * w0_b)
    y1 = PHI * jnp.sum(am_b * w1_b)

    act_z = act * (1.0 - active)
    bits = lax.bitcast_convert_type(act_z, jnp.int32)

    def bisect(_, carry):
        lo, hi = carry
        mid = lax.div(lo + hi, 2)
        cnt = jnp.sum((bits > mid).astype(jnp.int32))
        return jnp.where(cnt < KK, lo, mid + 1), jnp.where(cnt < KK, mid, hi)

    lo, hi = lax.fori_loop(0, 31, bisect, (jnp.int32(0), jnp.int32(2**31 - 1)))
    t = lo

    above = bits > t
    eq = bits == t
    n_above = jnp.sum(above.astype(jnp.int32))
    r = KK - n_above

    eqf = eq.astype(jnp.float32)
    li = lax.broadcasted_iota(jnp.int32, (128, 128), 0)
    lj = lax.broadcasted_iota(jnp.int32, (128, 128), 1)
    t_lane = (li < lj).astype(jnp.float32)
    lane_pref = jnp.dot(eqf, t_lane, preferred_element_type=jnp.float32)
    ri = lax.broadcasted_iota(jnp.int32, (ROWS, ROWS), 0)
    rj = lax.broadcasted_iota(jnp.int32, (ROWS, ROWS), 1)
    t_row = (rj < ri).astype(jnp.float32)
    row_cnt = jnp.sum(eqf, axis=1, keepdims=True)
    row_pref = jnp.dot(t_row, row_cnt, preferred_element_type=jnp.float32)
    rank = row_pref + lane_pref

    take_tie = eq & (rank < r.astype(jnp.float32)) & (t > 0)
    msk = (above | take_tie).astype(jnp.float32)
    s0 = jnp.sum(msk * w0_b)
    s1 = jnp.sum(msk * w1_b)

    yt0 = y_true_ref[0]
    yt1 = y_true_ref[1]
    pred_neq = (y0 >= y1) != (yt0 >= yt1)
    all_zero = (y0 == 0.0) & (y1 == 0.0)
    has_inactive = jnp.sum((active == 0.0).astype(jnp.int32)) > 0
    do_recruit = (pred_neq | all_zero) & has_inactive
    g = jnp.where(do_recruit, 1.0, 0.0)

    o0 = y0 + g * PHI * s0
    o1 = y1 + g * PHI * s1
    rr = lax.broadcasted_iota(jnp.int32, (8, 128), 0)
    cc = lax.broadcasted_iota(jnp.int32, (8, 128), 1)
    out_ref[...] = jnp.where(
        (rr == 0) & (cc == 0), o0, jnp.where((rr == 0) & (cc == 1), o1, 0.0))


@functools.partial(jax.jit, static_argnames=())
def _run(x, y_true, units, attn, W, active_units):
    act_col = pl.pallas_call(
        _act_body,
        grid=(MAX_NUNITS // BM,),
        in_specs=[
            pl.BlockSpec((1, N_DIMS), lambda i: (0, 0)),
            pl.BlockSpec((1, N_DIMS), lambda i: (0, 0)),
            pl.BlockSpec((BM, N_DIMS), lambda i: (i, 0)),
        ],
        out_specs=pl.BlockSpec((BM // 128, 128), lambda i: (i, 0)),
        out_shape=jax.ShapeDtypeStruct((ROWS, 128), jnp.float32),
    )(x.reshape(1, N_DIMS), attn.reshape(1, N_DIMS), units)

    act2 = act_col
    active2 = active_units.reshape(ROWS, 128)
    w0 = W[0].reshape(ROWS, 128)
    w1 = W[1].reshape(ROWS, 128)

    out2d = pl.pallas_call(
        _select_body,
        in_specs=[
            pl.BlockSpec(memory_space=pltpu.SMEM),
            pl.BlockSpec(memory_space=pltpu.VMEM),
            pl.BlockSpec(memory_space=pltpu.VMEM),
            pl.BlockSpec(memory_space=pltpu.VMEM),
            pl.BlockSpec(memory_space=pltpu.VMEM),
        ],
        out_shape=jax.ShapeDtypeStruct((8, 128), jnp.float32),
    )(y_true, act2, active2, w0, w1)
    return out2d[0, :2]


def kernel(x, epoch, signature, i, y_true, units, attn, W, active_units):
    del epoch, signature, i
    return _run(x, y_true, units, attn, W, active_units)


# speedup vs baseline: 1.0189x; 1.0189x over previous
import functools

import jax
import jax.numpy as jnp
from jax import lax
from jax.experimental import pallas as pl
from jax.experimental.pallas import tpu as pltpu

MAX_NUNITS = 65536
N_DIMS = 512
PHI = 1.5
KK = int(MAX_NUNITS * 0.01)

BM = 4096
ROWS = MAX_NUNITS // 128


def _act_body(x_ref, attn_ref, units_ref, act_ref):
    u = units_ref[...]
    d = x_ref[...] - u
    s = jnp.sum(attn_ref[...] * d * d, axis=1, keepdims=True)
    act_ref[...] = jnp.exp(-s).reshape(BM // 128, 128)


def _select_body(y_true_ref, act_ref, active_ref, w0_ref, w1_ref, out_ref):
    act = act_ref[...]
    active = active_ref[...]
    w0 = w0_ref[...]
    w1 = w1_ref[...]

    def _b(v):
        return v.astype(jnp.bfloat16).astype(jnp.float32)

    am_b = _b(act * active)
    w0_b = _b(w0)
    w1_b = _b(w1)
    y0 = PHI * jnp.sum(am_b * w0_b)
    y1 = PHI * jnp.sum(am_b * w1_b)

    act_z = act * (1.0 - active)
    bits = lax.bitcast_convert_type(act_z, jnp.int32)

    def bisect(_, carry):
        lo, hi = carry
        mid = lax.div(lo + hi, 2)
        cnt = jnp.sum((bits > mid).astype(jnp.int32))
        return jnp.where(cnt < KK, lo, mid + 1), jnp.where(cnt < KK, mid, hi)

    lo, hi = lax.fori_loop(0, 31, bisect, (jnp.int32(0), jnp.int32(2**31 - 1)))
    t = lo

    above = bits > t
    eq = bits == t
    n_above = jnp.sum(above.astype(jnp.int32))
    r = KK - n_above

    eqf = eq.astype(jnp.float32)
    li = lax.broadcasted_iota(jnp.int32, (128, 128), 0)
    lj = lax.broadcasted_iota(jnp.int32, (128, 128), 1)
    t_lane = (li < lj).astype(jnp.float32)
    lane_pref = jnp.dot(eqf, t_lane, preferred_element_type=jnp.float32)
    ri = lax.broadcasted_iota(jnp.int32, (ROWS, ROWS), 0)
    rj = lax.broadcasted_iota(jnp.int32, (ROWS, ROWS), 1)
    t_row = (rj < ri).astype(jnp.float32)
    row_cnt = jnp.sum(eqf, axis=1, keepdims=True)
    row_pref = jnp.dot(t_row, row_cnt, preferred_element_type=jnp.float32)
    rank = row_pref + lane_pref

    take_tie = eq & (rank < r.astype(jnp.float32)) & (t > 0)
    msk = (above | take_tie).astype(jnp.float32)
    s0 = jnp.sum(msk * w0_b)
    s1 = jnp.sum(msk * w1_b)

    yt0 = y_true_ref[0]
    yt1 = y_true_ref[1]
    pred_neq = (y0 >= y1) != (yt0 >= yt1)
    all_zero = (y0 == 0.0) & (y1 == 0.0)
    has_inactive = jnp.sum((active == 0.0).astype(jnp.int32)) > 0
    do_recruit = (pred_neq | all_zero) & has_inactive
    g = jnp.where(do_recruit, 1.0, 0.0)

    o0 = y0 + g * PHI * s0
    o1 = y1 + g * PHI * s1
    rr = lax.broadcasted_iota(jnp.int32, (8, 128), 0)
    cc = lax.broadcasted_iota(jnp.int32, (8, 128), 1)
    out_ref[...] = jnp.where(
        (rr == 0) & (cc == 0), o0, jnp.where((rr == 0) & (cc == 1), o1, 0.0))


@functools.partial(jax.jit, static_argnames=())
def _run(x, y_true, units, attn, W, active_units):
    act_col = pl.pallas_call(
        _act_body,
        grid=(MAX_NUNITS // BM,),
        in_specs=[
            pl.BlockSpec((1, N_DIMS), lambda i: (0, 0)),
            pl.BlockSpec((1, N_DIMS), lambda i: (0, 0)),
            pl.BlockSpec((BM, N_DIMS), lambda i: (i, 0)),
        ],
        out_specs=pl.BlockSpec((BM // 128, 128), lambda i: (i, 0)),
        out_shape=jax.ShapeDtypeStruct((ROWS, 128), jnp.float32),
    )(x.reshape(1, N_DIMS), attn.reshape(1, N_DIMS), units)

    act2 = act_col
    active2 = active_units.reshape(ROWS, 128)
    w0 = W[0].reshape(ROWS, 128)
    w1 = W[1].reshape(ROWS, 128)

    out2d = pl.pallas_call(
        _select_body,
        in_specs=[
            pl.BlockSpec(memory_space=pltpu.SMEM),
            pl.BlockSpec(memory_space=pltpu.VMEM),
            pl.BlockSpec(memory_space=pltpu.VMEM),
            pl.BlockSpec(memory_space=pltpu.VMEM),
            pl.BlockSpec(memory_space=pltpu.VMEM),
        ],
        out_shape=jax.ShapeDtypeStruct((8, 128), jnp.float32),
    )(y_true, act2, active2, w0, w1)
    return out2d[0, :2]


def kernel(x, epoch, signature, i, y_true, units, attn, W, active_units):
    del epoch, signature, i
    return _run(x, y_true, units, attn, W, active_units)
